# Initial kernel scaffold; baseline (speedup 1.0000x reference)
import jax
import jax.numpy as jnp
from jax.experimental import pallas as pl


def kernel(h):
    raise NotImplementedError("write your pallas kernel here")


# speedup vs baseline: 1.0022x; 1.0022x over previous
import jax
import jax.numpy as jnp
from jax.experimental import pallas as pl

_BATCH = 16384
_DIM = 2048
_BLOCK_ROWS = 1024


def _copy_body(h_ref, o_ref):
    o_ref[...] = h_ref[...]


def kernel(h):
    grid = (_BATCH // _BLOCK_ROWS,)
    return pl.pallas_call(
        _copy_body,
        grid=grid,
        in_specs=[pl.BlockSpec((_BLOCK_ROWS, _DIM), lambda i: (i, 0))],
        out_specs=pl.BlockSpec((_BLOCK_ROWS, _DIM), lambda i: (i, 0)),
        out_shape=jax.ShapeDtypeStruct((_BATCH, _DIM), jnp.float32),
    )(h)
